# Initial kernel scaffold; baseline (speedup 1.0000x reference)
import jax
import jax.numpy as jnp
from jax.experimental import pallas as pl


def kernel(h, r, t, ent_weight, rel_weight):
    raise NotImplementedError("write your pallas kernel here")


# speedup vs baseline: 1.2078x; 1.2078x over previous
import functools

import jax
import jax.numpy as jnp
from jax import lax
from jax.experimental import pallas as pl
from jax.experimental.pallas import tpu as pltpu
from jax.experimental.pallas import tpu_sc as plsc

L = 16
NUM_WORKERS = 32
CHUNK = 128
REV4 = [int("{:04b}".format(k)[::-1], 2) for k in range(L)]

_DNUMS = lax.GatherDimensionNumbers(
    offset_dims=(), collapsed_slice_dims=(0,), start_index_map=(0,))


def _shuf(v, perm):
    return lax.gather(v, perm, _DNUMS, slice_sizes=(1,),
                      mode=lax.GatherScatterMode.PROMISE_IN_BOUNDS)


def _build(B, D, NR):
    b_per_w = B // NUM_WORKERS
    n_chunks = b_per_w // CHUNK
    n_groups = b_per_w // L
    n_slices = D // L

    mesh = plsc.VectorSubcoreMesh(core_axis_name="c", subcore_axis_name="s")

    @functools.partial(
        pl.kernel,
        mesh=mesh,
        compiler_params=pltpu.CompilerParams(use_tc_tiling_on_sc=False),
        out_type=jax.ShapeDtypeStruct((B,), jnp.float32),
        scratch_types=[
            pltpu.VMEM((b_per_w,), jnp.int32),
            pltpu.VMEM((b_per_w,), jnp.int32),
            pltpu.VMEM((b_per_w,), jnp.int32),
            pltpu.VMEM((b_per_w, D), jnp.float32),
            pltpu.VMEM((b_per_w, D), jnp.float32),
            pltpu.VMEM((b_per_w, D), jnp.float32),
            pltpu.VMEM((b_per_w,), jnp.float32),
            pltpu.SemaphoreType.DMA,
        ],
    )
    def distmult(h_hbm, r_hbm, t_hbm, ent_hbm, rel_hbm, out_hbm,
                 h_idx, r_idx, t_idx, h_rows, r_rows, t_rows, out_v, sem):
        wid = lax.axis_index("s") * 2 + lax.axis_index("c")
        base = wid * b_per_w

        pltpu.sync_copy(h_hbm.at[pl.ds(base, b_per_w)], h_idx)
        pltpu.sync_copy(r_hbm.at[pl.ds(base, b_per_w)], r_idx)
        pltpu.sync_copy(t_hbm.at[pl.ds(base, b_per_w)], t_idx)

        for i in range(b_per_w // L):
            sl = pl.ds(i * L, L)
            v = r_idx[sl]
            r_idx[sl] = jnp.where(v >= NR, v - NR, v)

        copies = []
        for j in range(n_chunks):
            sl = pl.ds(j * CHUNK, CHUNK)
            copies.append(
                pltpu.async_copy(ent_hbm.at[h_idx.at[sl]], h_rows.at[sl], sem))
            copies.append(
                pltpu.async_copy(rel_hbm.at[r_idx.at[sl]], r_rows.at[sl], sem))
            copies.append(
                pltpu.async_copy(ent_hbm.at[t_idx.at[sl]], t_rows.at[sl], sem))
        for c in copies:
            c.wait()

        lane = lax.iota(jnp.int32, L)
        perms = {d: (lane ^ d).reshape(L, 1) for d in (8, 4, 2, 1)}
        masks = {d: (lane & d) == 0 for d in (8, 4, 2, 1)}

        def group_body(g, carry):
            row0 = g * L
            vecs = []
            for k in range(L):
                row = row0 + REV4[k]
                acc = None
                for s in range(n_slices):
                    dsl = pl.ds(s * L, L)
                    p = h_rows[row, dsl] * r_rows[row, dsl] * t_rows[row, dsl]
                    acc = p if acc is None else acc + p
                vecs.append(acc)
            for d in (8, 4, 2, 1):
                nxt = []
                for i in range(len(vecs) // 2):
                    u, v = vecs[2 * i], vecs[2 * i + 1]
                    m = jnp.where(masks[d], u, v)
                    n = jnp.where(masks[d], v, u)
                    nxt.append(m + _shuf(n, perms[d]))
                vecs = nxt
            out_v[pl.ds(row0, L)] = vecs[0]
            return carry

        lax.fori_loop(0, n_groups, group_body, 0)
        pltpu.sync_copy(out_v, out_hbm.at[pl.ds(base, b_per_w)])

    return distmult


def kernel(h, r, t, ent_weight, rel_weight):
    B = h.shape[0]
    D = ent_weight.shape[1]
    NR = rel_weight.shape[0]
    fn = _build(B, D, NR)
    return fn(h.astype(jnp.int32), r.astype(jnp.int32), t.astype(jnp.int32),
              ent_weight, rel_weight)


# speedup vs baseline: 1.4770x; 1.2229x over previous
import functools

import jax
import jax.numpy as jnp
from jax import lax
from jax.experimental import pallas as pl
from jax.experimental.pallas import tpu as pltpu
from jax.experimental.pallas import tpu_sc as plsc

L = 16
NUM_WORKERS = 32
PASS_ROWS = 256
CHUNK = 128
REV4 = [int("{:04b}".format(k)[::-1], 2) for k in range(L)]

_DNUMS = lax.GatherDimensionNumbers(
    offset_dims=(), collapsed_slice_dims=(0,), start_index_map=(0,))


def _shuf(v, perm):
    return lax.gather(v, perm, _DNUMS, slice_sizes=(1,),
                      mode=lax.GatherScatterMode.PROMISE_IN_BOUNDS)


def _build(B, D, NR2):
    b_per_w = B // NUM_WORKERS
    n_pass = b_per_w // PASS_ROWS
    g_per_pass = PASS_ROWS // L
    n_slices = D // L

    mesh = plsc.VectorSubcoreMesh(core_axis_name="c", subcore_axis_name="s")

    @functools.partial(
        pl.kernel,
        mesh=mesh,
        out_type=jax.ShapeDtypeStruct((B,), jnp.float32),
        scratch_types=[
            pltpu.VMEM((b_per_w,), jnp.int32),
            pltpu.VMEM((b_per_w,), jnp.int32),
            pltpu.VMEM((b_per_w,), jnp.int32),
            pltpu.VMEM((PASS_ROWS, D), jnp.float32),
            pltpu.VMEM((PASS_ROWS, 128), jnp.float32),
            pltpu.VMEM((PASS_ROWS, D), jnp.float32),
            pltpu.VMEM((b_per_w,), jnp.float32),
            pltpu.SemaphoreType.DMA,
        ],
    )
    def distmult(h_hbm, r_hbm, t_hbm, ent_hbm, rel_hbm, out_hbm,
                 h_idx, r_idx, t_idx, h_rows, r_rows, t_rows, out_v, sem):
        wid = lax.axis_index("s") * 2 + lax.axis_index("c")
        base = wid * b_per_w

        pltpu.sync_copy(h_hbm.at[pl.ds(base, b_per_w)], h_idx)
        pltpu.sync_copy(r_hbm.at[pl.ds(base, b_per_w)], r_idx)
        pltpu.sync_copy(t_hbm.at[pl.ds(base, b_per_w)], t_idx)

        lane = lax.iota(jnp.int32, L)
        perms = {d: (lane ^ d).reshape(L, 1) for d in (8, 4, 2, 1)}
        masks = {d: (lane & d) == 0 for d in (8, 4, 2, 1)}

        def run_pass(p, carry):
            p0 = p * PASS_ROWS

            for j in range(PASS_ROWS // CHUNK):
                pltpu.async_copy(
                    rel_hbm.at[r_idx.at[pl.ds(p0 + j * CHUNK, CHUNK)]],
                    r_rows.at[pl.ds(j * CHUNK, CHUNK)], sem)

            def issue_body(g, c):
                row0 = g * L
                hv = h_idx[pl.ds(p0 + row0, L)]
                tv = t_idx[pl.ds(p0 + row0, L)]
                for k in range(L):
                    pltpu.async_copy(
                        ent_hbm.at[pl.ds(hv[k], 1), :],
                        h_rows.at[pl.ds(row0 + k, 1), :], sem)
                    pltpu.async_copy(
                        ent_hbm.at[pl.ds(tv[k], 1), :],
                        t_rows.at[pl.ds(row0 + k, 1), :], sem)
                return c

            lax.fori_loop(0, g_per_pass, issue_body, 0)

            pltpu.make_async_copy(
                ent_hbm.at[pl.ds(0, PASS_ROWS), :], h_rows, sem).wait()
            pltpu.make_async_copy(
                ent_hbm.at[pl.ds(0, PASS_ROWS), :], t_rows, sem).wait()
            pltpu.make_async_copy(
                rel_hbm.at[pl.ds(0, PASS_ROWS), :], r_rows, sem).wait()

            def group_body(g, c):
                row0 = g * L
                vecs = []
                for k in range(L):
                    row = row0 + REV4[k]
                    acc = None
                    for s in range(n_slices):
                        dsl = pl.ds(s * L, L)
                        prod = (h_rows[row, dsl] * r_rows[row, dsl]
                                * t_rows[row, dsl])
                        acc = prod if acc is None else acc + prod
                    vecs.append(acc)
                for d in (8, 4, 2, 1):
                    nxt = []
                    for i in range(len(vecs) // 2):
                        u, v = vecs[2 * i], vecs[2 * i + 1]
                        m = jnp.where(masks[d], u, v)
                        n = jnp.where(masks[d], v, u)
                        nxt.append(m + _shuf(n, perms[d]))
                    vecs = nxt
                out_v[pl.ds(p0 + row0, L)] = vecs[0]
                return c

            lax.fori_loop(0, g_per_pass, group_body, 0)
            return carry

        lax.fori_loop(0, n_pass, run_pass, 0)
        pltpu.sync_copy(out_v, out_hbm.at[pl.ds(base, b_per_w)])

    return distmult


def kernel(h, r, t, ent_weight, rel_weight):
    B = h.shape[0]
    D = ent_weight.shape[1]
    NR = rel_weight.shape[0]
    rel128 = jnp.zeros((2 * NR, 128), jnp.float32)
    rel128 = rel128.at[:NR, :D].set(rel_weight).at[NR:, :D].set(rel_weight)
    fn = _build(B, D, 2 * NR)
    return fn(h.astype(jnp.int32), r.astype(jnp.int32), t.astype(jnp.int32),
              ent_weight, rel128)
